# Initial kernel scaffold; baseline (speedup 1.0000x reference)
import jax
import jax.numpy as jnp
from jax.experimental import pallas as pl


def kernel(H, LiMEs):
    raise NotImplementedError("write your pallas kernel here")


# speedup vs baseline: 10.3010x; 10.3010x over previous
import jax
import jax.numpy as jnp
from jax.experimental import pallas as pl
from jax.experimental.pallas import tpu as pltpu

_B = 4
_T = 2048
_D = 4096
_E = 64
_K = 8
_EPS = 1e-6
_TOK = 512


def _mix_body(hs_blk, hs_full, limes, out_ref, idxt_ref, scale_ref):
    i = pl.program_id(0)

    @pl.when(i == 0)
    def _():
        scale_ref[0, 0] = jnp.maximum(jnp.max(jnp.abs(hs_full[:, : _E])), _EPS)

    s = scale_ref[0, 0]
    x = hs_blk[:, : _E] / s

    iota = jax.lax.broadcasted_iota(jnp.int32, (_TOK, _E), 1)
    work = x
    vals = []
    idxs = []
    for _ in range(_K):
        mk = jnp.max(work, axis=-1, keepdims=True)
        cand = jnp.where(work == mk, iota, _E)
        ik = jnp.min(cand, axis=-1, keepdims=True)
        vals.append(mk)
        idxs.append(ik)
        work = jnp.where(iota == ik, -jnp.inf, work)
    topv = jnp.concatenate(vals, axis=1)
    topi = jnp.concatenate(idxs, axis=1)

    e = jnp.exp(topv - topv[:, 0:1])
    w = e / jnp.sum(e, axis=-1, keepdims=True)

    dense_w = jnp.zeros((_TOK, _E), jnp.float32)
    for k in range(_K):
        dense_w = dense_w + jnp.where(iota == topi[:, k : k + 1], w[:, k : k + 1], 0.0)

    out_ref[...] = jnp.dot(dense_w, limes[...], preferred_element_type=jnp.float32)
    idxt_ref[...] = topi.T


def kernel(H, LiMEs):
    BT = _B * _T
    H2 = H.reshape(BT, _D)
    grid = (BT // _TOK,)
    out, idx_t = pl.pallas_call(
        _mix_body,
        grid=grid,
        in_specs=[
            pl.BlockSpec((_TOK, 128), lambda i: (i, 0)),
            pl.BlockSpec((BT, 128), lambda i: (0, 0)),
            pl.BlockSpec((_E, _D), lambda i: (0, 0)),
        ],
        out_specs=[
            pl.BlockSpec((_TOK, _D), lambda i: (i, 0)),
            pl.BlockSpec((_K, _TOK), lambda i: (0, i)),
        ],
        out_shape=[
            jax.ShapeDtypeStruct((BT, _D), jnp.float32),
            jax.ShapeDtypeStruct((_K, BT), jnp.int32),
        ],
        scratch_shapes=[pltpu.SMEM((1, 1), jnp.float32)],
    )(H2, H2, LiMEs)
    p_mix = out.reshape(_B, _T, _D)
    topk_idx = idx_t.T.reshape(_B, _T, _K)
    return p_mix, topk_idx


# speedup vs baseline: 11.7270x; 1.1384x over previous
import jax
import jax.numpy as jnp
from jax.experimental import pallas as pl
from jax.experimental.pallas import tpu as pltpu

_B = 4
_T = 2048
_D = 4096
_E = 64
_K = 8
_EPS = 1e-6
_TOK = 512


def _mix_body(hs_blk, hs_full, limes, out_ref, idx_ref, scale_ref):
    i = pl.program_id(0)

    @pl.when(i == 0)
    def _():
        scale_ref[0, 0] = jnp.maximum(jnp.max(jnp.abs(hs_full[:, : _E])), _EPS)

    s = scale_ref[0, 0]
    x = hs_blk[:, : _E] / s

    riota = (
        (_E - 1) - jax.lax.broadcasted_iota(jnp.int32, (_TOK, _E), 1)
    ).astype(jnp.float32)
    work = x
    vals = []
    ridx = []
    for _ in range(_K):
        mk = jnp.max(work, axis=-1, keepdims=True)
        rk = jnp.max(jnp.where(work == mk, riota, -1.0), axis=-1, keepdims=True)
        vals.append(mk)
        ridx.append(rk)
        work = jnp.where(riota == rk, -jnp.inf, work)
    topv = jnp.concatenate(vals, axis=1)
    topr = jnp.concatenate(ridx, axis=1)

    e = jnp.exp(topv - topv[:, 0:1])
    w = e / jnp.sum(e, axis=-1, keepdims=True)

    dense_w = jnp.zeros((_TOK, _E), jnp.float32)
    for k in range(_K):
        dense_w = dense_w + jnp.where(riota == topr[:, k : k + 1], w[:, k : k + 1], 0.0)

    out_ref[...] = jnp.dot(dense_w, limes[...], preferred_element_type=jnp.float32)
    idx_ref[...] = (float(_E - 1) - topr).astype(jnp.int32)


def kernel(H, LiMEs):
    BT = _B * _T
    H2 = H.reshape(BT, _D)
    grid = (BT // _TOK,)
    out, idx = pl.pallas_call(
        _mix_body,
        grid=grid,
        in_specs=[
            pl.BlockSpec((_TOK, 128), lambda i: (i, 0)),
            pl.BlockSpec((BT, 128), lambda i: (0, 0)),
            pl.BlockSpec((_E, _D), lambda i: (0, 0)),
        ],
        out_specs=[
            pl.BlockSpec((_TOK, _D), lambda i: (i, 0)),
            pl.BlockSpec((_TOK, _K), lambda i: (i, 0)),
        ],
        out_shape=[
            jax.ShapeDtypeStruct((BT, _D), jnp.float32),
            jax.ShapeDtypeStruct((BT, _K), jnp.int32),
        ],
        scratch_shapes=[pltpu.SMEM((1, 1), jnp.float32)],
    )(H2, H2, LiMEs)
    p_mix = out.reshape(_B, _T, _D)
    topk_idx = idx.reshape(_B, _T, _K)
    return p_mix, topk_idx
